# baseline (speedup 1.0000x reference)
import functools

import jax
import jax.numpy as jnp
from jax import lax
from jax.experimental import pallas as pl
from jax.experimental.pallas import tpu as pltpu
from jax.experimental.pallas import tpu_sc as plsc

N_NODES = 10000
N_EDGES = 320000
D_FEAT = 128
D_EDGE = 16
D_OUT = 128

NUM_CORES = 2
NUM_SUBCORES = 16
NUM_WORKERS = NUM_CORES * NUM_SUBCORES
E_PER_W = N_EDGES // NUM_WORKERS
CHUNK = 400
N_CHUNKS = E_PER_W // CHUNK
N_PAIRS = N_CHUNKS // 2
SUB = 80
N_SUB = CHUNK // SUB

F_BLK = 16000


def _pq_body(nf_ref, w1_ref, w2_ref, p_ref, q_ref):
    nf = nf_ref[...]
    p_ref[...] = jnp.dot(nf, w1_ref[...], preferred_element_type=jnp.float32)
    q_ref[...] = jnp.dot(nf, w2_ref[...], preferred_element_type=jnp.float32)


def _fin_body(s_ref, ea_t_ref, w3_ref, b_ref, o_ref):
    o_ref[...] = (
        s_ref[...]
        + jax.lax.dot_general(
            ea_t_ref[...], w3_ref[...],
            dimension_numbers=(((0,), (0,)), ((), ())),
            preferred_element_type=jnp.float32)
        + b_ref[...]
    )


def _sc_gather(p_hbm, q_hbm, ei_hbm, s_hbm, row_v, col_v,
               buf0, buf1, sp0, sp1, sq0, sq1, sw0, sw1):
    wid = lax.axis_index("s") * NUM_CORES + lax.axis_index("c")
    base = wid * E_PER_W
    pltpu.sync_copy(ei_hbm.at[pl.ds(base, E_PER_W)], row_v)
    pltpu.sync_copy(ei_hbm.at[pl.ds(N_EDGES + base, E_PER_W)], col_v)

    def fire_p(j, buf, sem):
        for k in range(N_SUB):
            idx_off = j * CHUNK + k * SUB
            pltpu.async_copy(
                p_hbm.at[row_v.at[pl.ds(idx_off, SUB)]],
                buf.at[pl.ds(k * SUB, SUB)], sem)

    def fire_q(j, buf, sem):
        for k in range(N_SUB):
            idx_off = j * CHUNK + k * SUB
            pltpu.async_copy(
                q_hbm.at[col_v.at[pl.ds(idx_off, SUB)]],
                buf.at[pl.ds(k * SUB, SUB)], sem, add=True)

    def wait_gather(buf, sem):
        pltpu.make_async_copy(p_hbm.at[pl.ds(0, CHUNK)], buf, sem).wait()

    def fire_w(j, buf, sem):
        pltpu.make_async_copy(
            buf, s_hbm.at[pl.ds(base + j * CHUNK, CHUNK)], sem).start()

    def wait_w(buf, sem):
        pltpu.make_async_copy(
            buf, s_hbm.at[pl.ds(base, CHUNK)], sem).wait()

    fire_p(0, buf0, sp0)

    def pair_body(jj, carry):
        a = 2 * jj
        bb = a + 1

        wait_gather(buf0, sp0)
        fire_q(a, buf0, sq0)

        @pl.when(jj > 0)
        def _():
            wait_w(buf1, sw1)
        fire_p(bb, buf1, sp1)

        wait_gather(buf0, sq0)
        fire_w(a, buf0, sw0)

        wait_gather(buf1, sp1)
        fire_q(bb, buf1, sq1)

        @pl.when(jj < N_PAIRS - 1)
        def _():
            wait_w(buf0, sw0)
            fire_p(a + 2, buf0, sp0)

        wait_gather(buf1, sq1)
        fire_w(bb, buf1, sw1)
        return carry

    lax.fori_loop(0, N_PAIRS, pair_body, 0)
    if N_CHUNKS % 2:
        tail = N_CHUNKS - 1
        wait_w(buf0, sw0)
        fire_p(tail, buf0, sp0)
        wait_gather(buf0, sp0)
        fire_q(tail, buf0, sq0)
        wait_gather(buf0, sq0)
        fire_w(tail, buf0, sw0)
    wait_w(buf0, sw0)
    wait_w(buf1, sw1)


def kernel(node_feats, edge_index, edge_attr, W, b):
    ei2 = edge_index.astype(jnp.int32)
    ei = jnp.concatenate([ei2[0], ei2[1]])
    w1 = W[:D_FEAT]
    w2 = W[D_FEAT:2 * D_FEAT]
    w3 = W[2 * D_FEAT:]
    b2 = b.reshape(1, D_OUT)

    p, q = pl.pallas_call(
        _pq_body,
        out_shape=(
            jax.ShapeDtypeStruct((N_NODES, D_FEAT), jnp.float32),
            jax.ShapeDtypeStruct((N_NODES, D_FEAT), jnp.float32),
        ),
    )(node_feats, w1, w2)

    mesh = plsc.VectorSubcoreMesh(
        core_axis_name="c", subcore_axis_name="s",
        num_cores=NUM_CORES, num_subcores=NUM_SUBCORES)
    gather = functools.partial(
        pl.kernel,
        out_type=jax.ShapeDtypeStruct((N_EDGES, D_OUT), jnp.float32),
        mesh=mesh,
        scratch_types=[
            pltpu.VMEM((E_PER_W,), jnp.int32),
            pltpu.VMEM((E_PER_W,), jnp.int32),
            pltpu.VMEM((CHUNK, D_OUT), jnp.float32),
            pltpu.VMEM((CHUNK, D_OUT), jnp.float32),
            pltpu.SemaphoreType.DMA,
            pltpu.SemaphoreType.DMA,
            pltpu.SemaphoreType.DMA,
            pltpu.SemaphoreType.DMA,
            pltpu.SemaphoreType.DMA,
            pltpu.SemaphoreType.DMA,
        ],
    )(_sc_gather)
    s = gather(p, q, ei)

    return pl.pallas_call(
        _fin_body,
        grid=(N_EDGES // F_BLK,),
        in_specs=[
            pl.BlockSpec((F_BLK, D_OUT), lambda i: (i, 0)),
            pl.BlockSpec((D_EDGE, F_BLK), lambda i: (0, i)),
            pl.BlockSpec((D_EDGE, D_OUT), lambda i: (0, 0)),
            pl.BlockSpec((1, D_OUT), lambda i: (0, 0)),
        ],
        out_specs=pl.BlockSpec((F_BLK, D_OUT), lambda i: (i, 0)),
        out_shape=jax.ShapeDtypeStruct((N_EDGES, D_OUT), jnp.float32),
    )(s, edge_attr.T, w3, b2)


# speedup vs baseline: 1.0117x; 1.0117x over previous
import functools

import jax
import jax.numpy as jnp
from jax import lax
from jax.experimental import pallas as pl
from jax.experimental.pallas import tpu as pltpu
from jax.experimental.pallas import tpu_sc as plsc

N_NODES = 10000
N_EDGES = 320000
D_FEAT = 128
D_EDGE = 16
D_OUT = 128

NUM_CORES = 2
NUM_SUBCORES = 16
NUM_WORKERS = NUM_CORES * NUM_SUBCORES
N_SPLIT = 2
E_SPLIT = N_EDGES // N_SPLIT
E_PER_W = E_SPLIT // NUM_WORKERS
CHUNK = 200
N_CHUNKS = E_PER_W // CHUNK
N_PAIRS = N_CHUNKS // 2
SUB = 40
N_SUB = CHUNK // SUB

F_BLK = 16000
BLKS_PER_SPLIT = E_SPLIT // F_BLK


def _pq_body(nf_ref, w1_ref, w2_ref, p_ref, q_ref):
    nf = nf_ref[...]
    p_ref[...] = jnp.dot(nf, w1_ref[...], preferred_element_type=jnp.float32)
    q_ref[...] = jnp.dot(nf, w2_ref[...], preferred_element_type=jnp.float32)


def _fin_first_body(s_ref, ea_t_ref, w3_ref, b_ref, o_ref):
    o_ref[...] = (
        s_ref[...]
        + jax.lax.dot_general(
            ea_t_ref[...], w3_ref[...],
            dimension_numbers=(((0,), (0,)), ((), ())),
            preferred_element_type=jnp.float32)
        + b_ref[...]
    )


def _fin_body(prev_ref, s_ref, ea_t_ref, w3_ref, b_ref, o_ref):
    del prev_ref
    _fin_first_body(s_ref, ea_t_ref, w3_ref, b_ref, o_ref)


def _sc_gather(p_hbm, q_hbm, ei_hbm, s_hbm, row_v, col_v,
               buf0, buf1, sp0, sp1, sq0, sq1, sw0, sw1):
    wid = lax.axis_index("s") * NUM_CORES + lax.axis_index("c")
    base = wid * E_PER_W
    pltpu.sync_copy(ei_hbm.at[pl.ds(base, E_PER_W)], row_v)
    pltpu.sync_copy(ei_hbm.at[pl.ds(E_SPLIT + base, E_PER_W)], col_v)

    def fire_p(j, buf, sem):
        for k in range(N_SUB):
            idx_off = j * CHUNK + k * SUB
            pltpu.async_copy(
                p_hbm.at[row_v.at[pl.ds(idx_off, SUB)]],
                buf.at[pl.ds(k * SUB, SUB)], sem)

    def fire_q(j, buf, sem):
        for k in range(N_SUB):
            idx_off = j * CHUNK + k * SUB
            pltpu.async_copy(
                q_hbm.at[col_v.at[pl.ds(idx_off, SUB)]],
                buf.at[pl.ds(k * SUB, SUB)], sem, add=True)

    def wait_gather(buf, sem):
        pltpu.make_async_copy(p_hbm.at[pl.ds(0, CHUNK)], buf, sem).wait()

    def fire_w(j, buf, sem):
        pltpu.make_async_copy(
            buf, s_hbm.at[pl.ds(base + j * CHUNK, CHUNK)], sem).start()

    def wait_w(buf, sem):
        pltpu.make_async_copy(
            buf, s_hbm.at[pl.ds(base, CHUNK)], sem).wait()

    fire_p(0, buf0, sp0)

    def pair_body(jj, carry):
        a = 2 * jj
        bb = a + 1

        wait_gather(buf0, sp0)
        fire_q(a, buf0, sq0)

        @pl.when(jj > 0)
        def _():
            wait_w(buf1, sw1)
        fire_p(bb, buf1, sp1)

        wait_gather(buf0, sq0)
        fire_w(a, buf0, sw0)

        wait_gather(buf1, sp1)
        fire_q(bb, buf1, sq1)

        @pl.when(jj < N_PAIRS - 1)
        def _():
            wait_w(buf0, sw0)
            fire_p(a + 2, buf0, sp0)

        wait_gather(buf1, sq1)
        fire_w(bb, buf1, sw1)
        return carry

    lax.fori_loop(0, N_PAIRS, pair_body, 0)
    if N_CHUNKS % 2:
        tail = N_CHUNKS - 1
        wait_w(buf0, sw0)
        fire_p(tail, buf0, sp0)
        wait_gather(buf0, sp0)
        fire_q(tail, buf0, sq0)
        wait_gather(buf0, sq0)
        fire_w(tail, buf0, sw0)
    wait_w(buf0, sw0)
    wait_w(buf1, sw1)


def kernel(node_feats, edge_index, edge_attr, W, b):
    ei2 = edge_index.astype(jnp.int32)
    w1 = W[:D_FEAT]
    w2 = W[D_FEAT:2 * D_FEAT]
    w3 = W[2 * D_FEAT:]
    b2 = b.reshape(1, D_OUT)
    ea_t = edge_attr.T

    p, q = pl.pallas_call(
        _pq_body,
        out_shape=(
            jax.ShapeDtypeStruct((N_NODES, D_FEAT), jnp.float32),
            jax.ShapeDtypeStruct((N_NODES, D_FEAT), jnp.float32),
        ),
    )(node_feats, w1, w2)

    mesh = plsc.VectorSubcoreMesh(
        core_axis_name="c", subcore_axis_name="s",
        num_cores=NUM_CORES, num_subcores=NUM_SUBCORES)
    gather = functools.partial(
        pl.kernel,
        out_type=jax.ShapeDtypeStruct((E_SPLIT, D_OUT), jnp.float32),
        mesh=mesh,
        scratch_types=[
            pltpu.VMEM((E_PER_W,), jnp.int32),
            pltpu.VMEM((E_PER_W,), jnp.int32),
            pltpu.VMEM((CHUNK, D_OUT), jnp.float32),
            pltpu.VMEM((CHUNK, D_OUT), jnp.float32),
            pltpu.SemaphoreType.DMA,
            pltpu.SemaphoreType.DMA,
            pltpu.SemaphoreType.DMA,
            pltpu.SemaphoreType.DMA,
            pltpu.SemaphoreType.DMA,
            pltpu.SemaphoreType.DMA,
        ],
    )(_sc_gather)

    s_parts = []
    for sp in range(N_SPLIT):
        lo = sp * E_SPLIT
        ei = jnp.concatenate([ei2[0, lo:lo + E_SPLIT], ei2[1, lo:lo + E_SPLIT]])
        s_parts.append(gather(p, q, ei))

    out = None
    for sp in range(N_SPLIT):
        blk0 = sp * BLKS_PER_SPLIT
        data_specs = [
            pl.BlockSpec((F_BLK, D_OUT), lambda i: (i, 0)),
            pl.BlockSpec((D_EDGE, F_BLK), lambda i, b0=blk0: (0, b0 + i)),
            pl.BlockSpec((D_EDGE, D_OUT), lambda i: (0, 0)),
            pl.BlockSpec((1, D_OUT), lambda i: (0, 0)),
        ]
        out_spec = pl.BlockSpec((F_BLK, D_OUT), lambda i, b0=blk0: (b0 + i, 0))
        out_shape = jax.ShapeDtypeStruct((N_EDGES, D_OUT), jnp.float32)
        if sp == 0:
            out = pl.pallas_call(
                _fin_first_body,
                grid=(BLKS_PER_SPLIT,),
                in_specs=data_specs,
                out_specs=out_spec,
                out_shape=out_shape,
            )(s_parts[sp], ea_t, w3, b2)
        else:
            out = pl.pallas_call(
                _fin_body,
                grid=(BLKS_PER_SPLIT,),
                in_specs=[pl.BlockSpec(memory_space=pltpu.MemorySpace.HBM)]
                + data_specs,
                out_specs=out_spec,
                out_shape=out_shape,
                input_output_aliases={0: 0},
            )(out, s_parts[sp], ea_t, w3, b2)
    return out
